# baseline (speedup 1.0000x reference)
import functools

import jax
import jax.numpy as jnp
from jax import lax
from jax.experimental import pallas as pl
from jax.experimental.pallas import tpu as pltpu
from jax.experimental.pallas import tpu_sc as plsc

NUM_FRAME = 16384
TVS_DIM = 64
LANES = 16

NC, NS = 2, 16
NW = NC * NS
BPW = NUM_FRAME // NW
IDX_CHUNK = 128
NCHUNK = BPW // IDX_CHUNK


def _sc_body(z_hbm, means_hbm, stds_hbm, noise_hbm, out_hbm,
             idx_v, rows_m, rows_s, noise_v, sem):
    wid = lax.axis_index("s") * NC + lax.axis_index("c")
    base = wid * BPW

    pltpu.sync_copy(z_hbm.at[pl.ds(wid * NCHUNK, NCHUNK)], idx_v)
    pltpu.sync_copy(noise_hbm.at[pl.ds(base, BPW)], noise_v)

    copies = []
    for j in range(NCHUNK):
        copies.append(pltpu.async_copy(
            means_hbm.at[idx_v.at[j]],
            rows_m.at[pl.ds(j * IDX_CHUNK, IDX_CHUNK)], sem))
        copies.append(pltpu.async_copy(
            stds_hbm.at[idx_v.at[j]],
            rows_s.at[pl.ds(j * IDX_CHUNK, IDX_CHUNK)], sem))
    for c in copies:
        c.wait()

    def group_body(g, carry):
        nz16 = noise_v[pl.ds(g * LANES, LANES)]
        for r in range(LANES):
            i = g * LANES + r
            nz = nz16[r]
            for c in range(TVS_DIM // LANES):
                sl = pl.ds(c * LANES, LANES)
                m = rows_m[i, sl]
                s = rows_s[i, sl]
                rows_m[i, sl] = jnp.clip(m + nz * s, -1.0, 1.0)
        return carry

    lax.fori_loop(0, BPW // LANES, group_body, 0)

    pltpu.sync_copy(rows_m, out_hbm.at[pl.ds(base, BPW)])


@jax.jit
def kernel(z, target_means, target_stds, noise):
    z2 = z.astype(jnp.int32).reshape(NW * NCHUNK, IDX_CHUNK)
    noise1 = noise.reshape(NUM_FRAME)

    mesh = plsc.VectorSubcoreMesh(
        core_axis_name="c", subcore_axis_name="s",
        num_cores=NC, num_subcores=NS)
    run = pl.kernel(
        _sc_body,
        mesh=mesh,
        out_type=jax.ShapeDtypeStruct((NUM_FRAME, TVS_DIM), jnp.float32),
        scratch_types=[
            pltpu.VMEM((NCHUNK, IDX_CHUNK), jnp.int32),
            pltpu.VMEM((BPW, TVS_DIM), jnp.float32),
            pltpu.VMEM((BPW, TVS_DIM), jnp.float32),
            pltpu.VMEM((BPW,), jnp.float32),
            pltpu.SemaphoreType.DMA,
        ],
        compiler_params=pltpu.CompilerParams(use_tc_tiling_on_sc=False),
    )
    return run(z2, target_means, target_stds, noise1)
